# baseline (device time: 70141 ns/iter reference)
import jax
import jax.numpy as jnp
from jax import lax
from jax.experimental import pallas as pl
from jax.experimental.pallas import tpu as pltpu

NT = 4


def kernel(x, assign, W1, W2):
    t, d = x.shape
    e_loc, _, f = W1.shape
    th = t // 2
    tt = th // NT

    my_y_out = lax.axis_index("y")
    ge_all = jnp.concatenate([
        my_y_out * e_loc + jnp.arange(e_loc),
        (1 - my_y_out) * e_loc + jnp.arange(e_loc),
    ])
    perm = jnp.argsort(assign)
    xs = x[perm]
    oh_all = (assign[perm][:, None] == ge_all[None, :]).astype(jnp.bfloat16)

    def body(x_ref, oh_ref, w1_hbm, w2_hbm, out_ref,
             s1, s2, w1b, w2b,
             xsend, ohsend, xpeer, ohpeer, rsend, rret, osend, accr,
             sems_xs, sems_xr, sems_as, sems_ar,
             sems_rs, sems_rr, sems_os, sems_or, sems_st, local_sems):
        my_x = lax.axis_index("x")
        my_y = lax.axis_index("y")
        ypeer_id = (my_x, 1 - my_y)
        xpeer_id = (1 - my_x, my_y)

        def gsl(k):
            return pl.ds(pl.multiple_of((2 * k + my_x) * tt, tt), tt)

        barrier = pltpu.get_barrier_semaphore()
        for nbr in (ypeer_id, xpeer_id):
            pl.semaphore_signal(barrier, inc=1, device_id=nbr,
                                device_id_type=pl.DeviceIdType.MESH)
        pl.semaphore_wait(barrier, 2)

        stages = (s1, s2)

        def w_chunk(c):
            e, half, is_w2 = c // 4, (c // 2) % 2, c % 2
            dsl = pl.ds(half * d, d)
            if is_w2:
                return (w2_hbm.at[e, dsl, :], lambda v: w2b.__setitem__(
                    (e, dsl, slice(None)), v))
            return (w1_hbm.at[e, :, dsl], lambda v: w1b.__setitem__(
                (e, slice(None), dsl), v))

        def w_start(c):
            src, _ = w_chunk(c)
            cp = pltpu.make_async_copy(src, stages[c % 2],
                                       local_sems.at[c % 2])
            cp.start()
            return cp

        pending = [w_start(0), w_start(1)]

        for k in range(NT):
            ohsend[pl.ds(k * tt, tt), :] = oh_ref[gsl(k), 2:4]
        rdma_a = pltpu.make_async_remote_copy(
            src_ref=ohsend, dst_ref=ohpeer,
            send_sem=sems_as.at[0], recv_sem=sems_ar.at[0],
            device_id=ypeer_id, device_id_type=pl.DeviceIdType.MESH)
        rdma_a.start()

        rdma_x = []
        for k in range(NT):
            sl = pl.ds(k * tt, tt)
            xsend[sl, :] = x_ref[gsl(k), :].astype(jnp.bfloat16)
            r = pltpu.make_async_remote_copy(
                src_ref=xsend.at[sl], dst_ref=xpeer.at[sl],
                send_sem=sems_xs.at[k], recv_sem=sems_xr.at[k],
                device_id=ypeer_id, device_id_type=pl.DeviceIdType.MESH)
            r.start()
            rdma_x.append(r)

        for c in range(8):
            pending[c % 2].wait()
            _, store = w_chunk(c)
            store(stages[c % 2][...].astype(jnp.bfloat16))
            if c + 2 < 8:
                pending[c % 2] = w_start(c + 2)

        def contrib_to(dst_sl, dst_ref, xblk, ohblk, extra=None):
            accr[...] = (jnp.zeros((tt, d), jnp.float32)
                         if extra is None else extra)
            for e in range(e_loc):
                npos = jnp.sum(ohblk[:, e:e + 1].astype(jnp.float32))

                @pl.when(npos > 0)
                def _():
                    xm = xblk * ohblk[:, e:e + 1]
                    h = jnp.dot(xm, w1b[e],
                                preferred_element_type=jnp.float32)
                    h = jnp.maximum(h, 0.0).astype(jnp.bfloat16)
                    accr[...] += jnp.dot(h, w2b[e],
                                         preferred_element_type=jnp.float32)
            dst_ref[dst_sl, :] = accr[...].astype(jnp.bfloat16)

        rdma_a.wait()
        rdma_r = []
        for k in range(NT):
            sl = pl.ds(k * tt, tt)
            rdma_x[k].wait()
            contrib_to(sl, rsend, xpeer[sl, :], ohpeer[sl, :])
            r = pltpu.make_async_remote_copy(
                src_ref=rsend.at[sl], dst_ref=rret.at[sl],
                send_sem=sems_rs.at[k], recv_sem=sems_rr.at[k],
                device_id=ypeer_id, device_id_type=pl.DeviceIdType.MESH)
            r.start()
            rdma_r.append(r)

        rdma_o, stores = [], []
        for k in range(NT):
            sl = pl.ds(k * tt, tt)
            rdma_r[k].wait()
            contrib_to(sl, osend, xsend[sl, :], oh_ref[gsl(k), 0:2],
                       extra=rret[sl, :].astype(jnp.float32))
            st = pltpu.make_async_copy(osend.at[sl], out_ref.at[gsl(k)],
                                       sems_st.at[k])
            st.start()
            stores.append(st)
            r = pltpu.make_async_remote_copy(
                src_ref=osend.at[sl], dst_ref=out_ref.at[gsl(k)],
                send_sem=sems_os.at[k], recv_sem=sems_or.at[k],
                device_id=xpeer_id, device_id_type=pl.DeviceIdType.MESH)
            r.start()
            rdma_o.append(r)

        for k in range(NT):
            stores[k].wait()
            rdma_o[k].wait()

    outs = pl.pallas_call(
        body,
        out_shape=jax.ShapeDtypeStruct((t, d), jnp.bfloat16),
        in_specs=[
            pl.BlockSpec(memory_space=pltpu.VMEM),
            pl.BlockSpec(memory_space=pltpu.VMEM),
            pl.BlockSpec(memory_space=pl.ANY),
            pl.BlockSpec(memory_space=pl.ANY),
        ],
        out_specs=pl.BlockSpec(memory_space=pl.ANY),
        scratch_shapes=[
            pltpu.VMEM((d, d), jnp.float32),
            pltpu.VMEM((d, d), jnp.float32),
            pltpu.VMEM((e_loc, d, f), jnp.bfloat16),
            pltpu.VMEM((e_loc, f, d), jnp.bfloat16),
            pltpu.VMEM((th, d), jnp.bfloat16),
            pltpu.VMEM((th, e_loc), jnp.bfloat16),
            pltpu.VMEM((th, d), jnp.bfloat16),
            pltpu.VMEM((th, e_loc), jnp.bfloat16),
            pltpu.VMEM((th, d), jnp.bfloat16),
            pltpu.VMEM((th, d), jnp.bfloat16),
            pltpu.VMEM((th, d), jnp.bfloat16),
            pltpu.VMEM((tt, d), jnp.float32),
            pltpu.SemaphoreType.DMA((NT,)),
            pltpu.SemaphoreType.DMA((NT,)),
            pltpu.SemaphoreType.DMA((1,)),
            pltpu.SemaphoreType.DMA((1,)),
            pltpu.SemaphoreType.DMA((NT,)),
            pltpu.SemaphoreType.DMA((NT,)),
            pltpu.SemaphoreType.DMA((NT,)),
            pltpu.SemaphoreType.DMA((NT,)),
            pltpu.SemaphoreType.DMA((NT,)),
            pltpu.SemaphoreType.DMA((2,)),
        ],
        compiler_params=pltpu.CompilerParams(
            collective_id=0, vmem_limit_bytes=60 * 1024 * 1024),
    )(xs, oh_all, W1, W2)

    return outs[jnp.argsort(perm)]


# device time: 48197 ns/iter; 1.4553x vs baseline; 1.4553x over previous
import jax
import jax.numpy as jnp
from jax import lax
from jax.experimental import pallas as pl
from jax.experimental.pallas import tpu as pltpu

NT = 4


def kernel(x, assign, W1, W2):
    t, d = x.shape
    e_loc, _, f = W1.shape
    th = t // 2
    tt = th // NT

    my_y_out = lax.axis_index("y")
    ge_all = jnp.concatenate([
        my_y_out * e_loc + jnp.arange(e_loc),
        (1 - my_y_out) * e_loc + jnp.arange(e_loc),
    ])
    oh_all = (assign[:, None] == ge_all[None, :]).astype(jnp.bfloat16)

    def body(x_ref, oh_ref, w1_hbm, w2_hbm, out_ref,
             s1, s2, w1b, w2b,
             xsend, ohsend, xpeer, ohpeer, rsend, rret, osend,
             sems_xs, sems_xr, sems_as, sems_ar,
             sems_rs, sems_rr, sems_os, sems_or, sems_st, local_sems):
        my_x = lax.axis_index("x")
        my_y = lax.axis_index("y")
        ypeer_id = (my_x, 1 - my_y)
        xpeer_id = (1 - my_x, my_y)

        def gsl(k):
            return pl.ds(pl.multiple_of(my_x * th + k * tt, tt), tt)

        barrier = pltpu.get_barrier_semaphore()
        for nbr in (ypeer_id, xpeer_id):
            pl.semaphore_signal(barrier, inc=1, device_id=nbr,
                                device_id_type=pl.DeviceIdType.MESH)
        pl.semaphore_wait(barrier, 2)

        stages = (s1, s2)

        def w_chunk(c):
            e, half, is_w2 = c // 4, (c // 2) % 2, c % 2
            dsl = pl.ds(half * d, d)
            if is_w2:
                return (w2_hbm.at[e, dsl, :], lambda v: w2b.__setitem__(
                    (e, dsl, slice(None)), v))
            return (w1_hbm.at[e, :, dsl], lambda v: w1b.__setitem__(
                (e, slice(None), dsl), v))

        def w_start(c):
            src, _ = w_chunk(c)
            cp = pltpu.make_async_copy(src, stages[c % 2],
                                       local_sems.at[c % 2])
            cp.start()
            return cp

        pending = [w_start(0), w_start(1)]

        for k in range(NT):
            ohsend[pl.ds(k * tt, tt), :] = oh_ref[gsl(k), 2:4]
        rdma_a = pltpu.make_async_remote_copy(
            src_ref=ohsend, dst_ref=ohpeer,
            send_sem=sems_as.at[0], recv_sem=sems_ar.at[0],
            device_id=ypeer_id, device_id_type=pl.DeviceIdType.MESH)
        rdma_a.start()

        rdma_x = []
        for k in range(NT):
            sl = pl.ds(k * tt, tt)
            xsend[sl, :] = x_ref[gsl(k), :].astype(jnp.bfloat16)
            r = pltpu.make_async_remote_copy(
                src_ref=xsend.at[sl], dst_ref=xpeer.at[sl],
                send_sem=sems_xs.at[k], recv_sem=sems_xr.at[k],
                device_id=ypeer_id, device_id_type=pl.DeviceIdType.MESH)
            r.start()
            rdma_x.append(r)

        for c in range(8):
            pending[c % 2].wait()
            _, store = w_chunk(c)
            store(stages[c % 2][...].astype(jnp.bfloat16))
            if c + 2 < 8:
                pending[c % 2] = w_start(c + 2)

        def contrib_to(dst_sl, dst_ref, xblk, ohblk, extra=None):
            acc = extra
            for e in range(e_loc):
                xm = xblk * ohblk[:, e:e + 1]
                h = jnp.dot(xm, w1b[e], preferred_element_type=jnp.float32)
                h = jnp.maximum(h, 0.0).astype(jnp.bfloat16)
                o = jnp.dot(h, w2b[e], preferred_element_type=jnp.float32)
                acc = o if acc is None else acc + o
            dst_ref[dst_sl, :] = acc.astype(jnp.bfloat16)

        rdma_a.wait()
        rdma_r = []
        for k in range(NT):
            sl = pl.ds(k * tt, tt)
            rdma_x[k].wait()
            contrib_to(sl, rsend, xpeer[sl, :], ohpeer[sl, :])
            r = pltpu.make_async_remote_copy(
                src_ref=rsend.at[sl], dst_ref=rret.at[sl],
                send_sem=sems_rs.at[k], recv_sem=sems_rr.at[k],
                device_id=ypeer_id, device_id_type=pl.DeviceIdType.MESH)
            r.start()
            rdma_r.append(r)

        rdma_o, stores = [], []
        for k in range(NT):
            sl = pl.ds(k * tt, tt)
            rdma_r[k].wait()
            contrib_to(sl, osend, xsend[sl, :], oh_ref[gsl(k), 0:2],
                       extra=rret[sl, :].astype(jnp.float32))
            st = pltpu.make_async_copy(osend.at[sl], out_ref.at[gsl(k)],
                                       sems_st.at[k])
            st.start()
            stores.append(st)
            r = pltpu.make_async_remote_copy(
                src_ref=osend.at[sl], dst_ref=out_ref.at[gsl(k)],
                send_sem=sems_os.at[k], recv_sem=sems_or.at[k],
                device_id=xpeer_id, device_id_type=pl.DeviceIdType.MESH)
            r.start()
            rdma_o.append(r)

        for k in range(NT):
            stores[k].wait()
            rdma_o[k].wait()

    return pl.pallas_call(
        body,
        out_shape=jax.ShapeDtypeStruct((t, d), jnp.bfloat16),
        in_specs=[
            pl.BlockSpec(memory_space=pltpu.VMEM),
            pl.BlockSpec(memory_space=pltpu.VMEM),
            pl.BlockSpec(memory_space=pl.ANY),
            pl.BlockSpec(memory_space=pl.ANY),
        ],
        out_specs=pl.BlockSpec(memory_space=pl.ANY),
        scratch_shapes=[
            pltpu.VMEM((d, d), jnp.float32),
            pltpu.VMEM((d, d), jnp.float32),
            pltpu.VMEM((e_loc, d, f), jnp.bfloat16),
            pltpu.VMEM((e_loc, f, d), jnp.bfloat16),
            pltpu.VMEM((th, d), jnp.bfloat16),
            pltpu.VMEM((th, e_loc), jnp.bfloat16),
            pltpu.VMEM((th, d), jnp.bfloat16),
            pltpu.VMEM((th, e_loc), jnp.bfloat16),
            pltpu.VMEM((th, d), jnp.bfloat16),
            pltpu.VMEM((th, d), jnp.bfloat16),
            pltpu.VMEM((th, d), jnp.bfloat16),
            pltpu.SemaphoreType.DMA((NT,)),
            pltpu.SemaphoreType.DMA((NT,)),
            pltpu.SemaphoreType.DMA((1,)),
            pltpu.SemaphoreType.DMA((1,)),
            pltpu.SemaphoreType.DMA((NT,)),
            pltpu.SemaphoreType.DMA((NT,)),
            pltpu.SemaphoreType.DMA((NT,)),
            pltpu.SemaphoreType.DMA((NT,)),
            pltpu.SemaphoreType.DMA((NT,)),
            pltpu.SemaphoreType.DMA((2,)),
        ],
        compiler_params=pltpu.CompilerParams(
            collective_id=0, vmem_limit_bytes=60 * 1024 * 1024),
    )(x, oh_all, W1, W2)
